# baseline (speedup 1.0000x reference)
import jax
import jax.numpy as jnp
from jax import lax
from jax.experimental import pallas as pl
from jax.experimental.pallas import tpu as pltpu
from jax.experimental.pallas import tpu_sc as plsc

_N = 10000
_NP = 10240
_E = 320000
_D = 128
_NC = 2
_NS = 16
_NW = _NC * _NS
_C1 = 80
_EPW = _E // _NW
_NCH = _EPW // _C1
_RW = 80
_NR = _E // _RW
_RPW = _NR // _NW
_K = 5
_CP4 = _NP // _NW

_mesh = plsc.VectorSubcoreMesh(core_axis_name="c", subcore_axis_name="s")
_params = pltpu.CompilerParams(needs_layout_passes=False)


def _p1_body(se_hbm, hcu_hbm, re_hbm, ws_hbm, b_hbm,
             ow_hbm, ob_hbm, exps_hbm,
             se_b0, vi_b0, vj_b0, rel_b0, viu_b0, vju_b0, r_b0,
             se_b1, vi_b1, vj_b1, rel_b1, viu_b1, vju_b1, r_b1,
             wsv, bv, owv, obv, acc_buf, ex_b0, ex_b1, sem0, sem1):
    cid = lax.axis_index("c")
    sid = lax.axis_index("s")
    wid = sid * _NC + cid
    pltpu.sync_copy(ws_hbm, wsv)
    pltpu.sync_copy(b_hbm, bv)
    pltpu.sync_copy(ow_hbm, owv)
    pltpu.sync_copy(ob_hbm, obv)
    base0 = wid * _EPW
    slots = ((se_b0, vi_b0, vj_b0, rel_b0, viu_b0, vju_b0, r_b0, ex_b0, sem0),
             (se_b1, vi_b1, vj_b1, rel_b1, viu_b1, vju_b1, r_b1, ex_b1, sem1))

    def stage_fire(k, slot):
        se_b, vi_b, vj_b, rel_b, viu_b, vju_b, r_b, _, sem = slot
        base = base0 + k * _C1
        pltpu.sync_copy(se_hbm.at[pl.ds(base * 8, _C1 * 8)], se_b)
        for gg in range(_C1 // 16):
            dsg = pl.ds(gg * 16, 16)
            flat = (lax.iota(jnp.int32, 16) + (gg * 16)) * 8
            vi_b[dsg] = plsc.load_gather(se_b, [flat + 1])
            vj_b[dsg] = plsc.load_gather(se_b, [flat + 2])
            rel_b[dsg] = plsc.load_gather(se_b, [flat + 3])
        pltpu.async_copy(hcu_hbm.at[vi_b], viu_b, sem)
        pltpu.async_copy(hcu_hbm.at[vj_b], vju_b, sem)
        pltpu.async_copy(re_hbm.at[rel_b], r_b, sem)

    def drain(slot):
        _, vi_b, vj_b, rel_b, viu_b, vju_b, r_b, _, sem = slot
        pltpu.make_async_copy(hcu_hbm.at[vi_b], viu_b, sem).wait()
        pltpu.make_async_copy(hcu_hbm.at[vj_b], vju_b, sem).wait()
        pltpu.make_async_copy(re_hbm.at[rel_b], r_b, sem).wait()

    def compute_out(k, slot):
        _, _, _, _, viu_b, vju_b, r_b, ex_b, _ = slot
        base = base0 + k * _C1
        for g in range(_D // 16):
            dsl = pl.ds(g * 16, 16)
            dsl2 = pl.ds(_D + g * 16, 16)
            w0 = wsv[0, dsl]
            w1 = wsv[1, dsl]
            w2 = wsv[2, dsl]
            w3 = wsv[3, dsl]
            w4 = wsv[4, dsl]
            w5 = wsv[5, dsl]
            w6 = wsv[6, dsl]
            w7 = wsv[7, dsl]
            bb = bv[dsl]
            ow = owv[dsl]
            ob = obv[dsl]

            def ebody(e, g=g, dsl=dsl, dsl2=dsl2, viu_b=viu_b, vju_b=vju_b,
                      r_b=r_b, w0=w0, w1=w1, w2=w2, w3=w3, w4=w4, w5=w5,
                      w6=w6, w7=w7, bb=bb, ow=ow, ob=ob):
                a = viu_b[e, dsl]
                au = viu_b[e, dsl2]
                c = vju_b[e, dsl]
                cu = vju_b[e, dsl2]
                r = r_b[e, dsl]
                p = a * (w0 + w1 * r) + au * (w4 + w5 * r)
                q = a * (w2 + w3 * r) + au * (w6 + w7 * r)
                t = c * p + cu * q + bb
                t = jnp.maximum(t, 0.0) * ow + ob
                sl = pl.ds(e * 16, 16)
                if g == 0:
                    acc_buf[sl] = t
                else:
                    acc_buf[sl] = acc_buf[sl] + t

            plsc.parallel_loop(0, _C1, unroll=4)(ebody)
        for gg in range(_C1 // 16):
            rows16 = (lax.iota(jnp.int32, 16) + (gg * 16)) * 16
            tot = jnp.zeros((16,), jnp.float32)
            for j in range(16):
                tot = tot + plsc.load_gather(acc_buf, [rows16 + j])
            ex_b[pl.ds(gg * 16, 16)] = jnp.exp(tot)
        pltpu.sync_copy(ex_b, exps_hbm.at[pl.ds(base, _C1)])

    stage_fire(0, slots[0])

    @pl.loop(0, _NCH // 2)
    def _pair(p):
        k0 = 2 * p
        stage_fire(k0 + 1, slots[1])
        drain(slots[0])
        compute_out(k0, slots[0])
        stage_fire(k0 + 2, slots[0])
        drain(slots[1])
        compute_out(k0 + 1, slots[1])

    drain(slots[0])
    compute_out(_NCH - 1, slots[0])


def _p2_body(exps_hbm, ivi_hbm, zeros_hbm, d0_hbm, d1_hbm,
             idx_bs, val_bs, shared, semA, semB):
    cid = lax.axis_index("c")
    sid = lax.axis_index("s")
    wid = sid * _NC + cid

    @pl.when(sid == 0)
    def _zero():
        pltpu.sync_copy(zeros_hbm, shared)

    plsc.subcore_barrier()

    @pl.loop(0, _RPW // _K)
    def _blk(bi):
        r0 = wid * _RPW + bi * _K
        sc = []
        for j in range(_K):
            base = (r0 + j) * _RW
            sc.append(pltpu.async_copy(ivi_hbm.at[pl.ds(base, _RW)],
                                       idx_bs[j], semA))
            sc.append(pltpu.async_copy(exps_hbm.at[pl.ds(base, _RW)],
                                       val_bs[j], semA))
        for cp in sc:
            cp.wait()
        cps = [pltpu.async_copy(val_bs[j], shared.at[idx_bs[j]], semB,
                                add=True) for j in range(_K)]
        for cp in cps:
            cp.wait()

    plsc.subcore_barrier()

    @pl.when(sid == 0)
    def _out():
        @pl.when(cid == 0)
        def _c0():
            pltpu.sync_copy(shared, d0_hbm)

        @pl.when(cid == 1)
        def _c1():
            pltpu.sync_copy(shared, d1_hbm)


def _p3_body(exps_hbm, ivi_hbm, vi_hbm, vj_hbm, den_hbm, inp_hbm,
             zeros_hbm, o0_hbm, o1_hbm,
             ivi_bs, vi_bs, vj_bs, ex_bs, den_bs, inp_bs, att_bs,
             shared, semA, semB):
    cid = lax.axis_index("c")
    sid = lax.axis_index("s")
    wid = sid * _NC + cid

    @pl.when(sid == 0)
    def _zero():
        pltpu.sync_copy(zeros_hbm, shared)

    plsc.subcore_barrier()

    @pl.loop(0, _RPW // _K)
    def _blk(bi):
        r0 = wid * _RPW + bi * _K
        sc = []
        for j in range(_K):
            base = (r0 + j) * _RW
            sc.append(pltpu.async_copy(ivi_hbm.at[pl.ds(base, _RW)],
                                       ivi_bs[j], semA))
            sc.append(pltpu.async_copy(vi_hbm.at[pl.ds(base, _RW)],
                                       vi_bs[j], semA))
            sc.append(pltpu.async_copy(vj_hbm.at[pl.ds(base, _RW)],
                                       vj_bs[j], semA))
            sc.append(pltpu.async_copy(exps_hbm.at[pl.ds(base, _RW)],
                                       ex_bs[j], semA))
        for cp in sc:
            cp.wait()
        gcps = []
        for j in range(_K):
            gcps.append(pltpu.async_copy(den_hbm.at[ivi_bs[j]], den_bs[j],
                                         semA))
            gcps.append(pltpu.async_copy(inp_hbm.at[vi_bs[j]], inp_bs[j],
                                         semA))
        for cp in gcps:
            cp.wait()
        for j in range(_K):
            for g in range(_RW // 16):
                dsl = pl.ds(g * 16, 16)
                att_bs[j][dsl] = (ex_bs[j][dsl] * inp_bs[j][dsl]
                                  / den_bs[j][dsl])
        scps = [pltpu.async_copy(att_bs[j], shared.at[vj_bs[j]], semB,
                                 add=True) for j in range(_K)]
        for cp in scps:
            cp.wait()

    plsc.subcore_barrier()

    @pl.when(sid == 0)
    def _out():
        @pl.when(cid == 0)
        def _c0():
            pltpu.sync_copy(shared, o0_hbm)

        @pl.when(cid == 1)
        def _c1():
            pltpu.sync_copy(shared, o1_hbm)


def _p4_body(o0_hbm, o1_hbm, out_hbm, a_b, b_b, s_b):
    cid = lax.axis_index("c")
    sid = lax.axis_index("s")
    wid = sid * _NC + cid
    base = wid * _CP4
    pltpu.sync_copy(o0_hbm.at[pl.ds(base, _CP4)], a_b)
    pltpu.sync_copy(o1_hbm.at[pl.ds(base, _CP4)], b_b)
    for g in range(_CP4 // 16):
        sl = pl.ds(g * 16, 16)
        s_b[sl] = a_b[sl] + b_b[sl]
    pltpu.sync_copy(s_b, out_hbm.at[pl.ds(base, _CP4)])


_phase1 = pl.kernel(
    _p1_body,
    out_type=jax.ShapeDtypeStruct((_E,), jnp.float32),
    mesh=_mesh,
    compiler_params=_params,
    scratch_types=[
        pltpu.VMEM((_C1 * 8,), jnp.int32),
        pltpu.VMEM((_C1,), jnp.int32),
        pltpu.VMEM((_C1,), jnp.int32),
        pltpu.VMEM((_C1,), jnp.int32),
        pltpu.VMEM((_C1, 2 * _D), jnp.float32),
        pltpu.VMEM((_C1, 2 * _D), jnp.float32),
        pltpu.VMEM((_C1, _D), jnp.float32),
        pltpu.VMEM((_C1 * 8,), jnp.int32),
        pltpu.VMEM((_C1,), jnp.int32),
        pltpu.VMEM((_C1,), jnp.int32),
        pltpu.VMEM((_C1,), jnp.int32),
        pltpu.VMEM((_C1, 2 * _D), jnp.float32),
        pltpu.VMEM((_C1, 2 * _D), jnp.float32),
        pltpu.VMEM((_C1, _D), jnp.float32),
        pltpu.VMEM((8, _D), jnp.float32),
        pltpu.VMEM((_D,), jnp.float32),
        pltpu.VMEM((_D,), jnp.float32),
        pltpu.VMEM((_D,), jnp.float32),
        pltpu.VMEM((_C1 * 16,), jnp.float32),
        pltpu.VMEM((_C1,), jnp.float32),
        pltpu.VMEM((_C1,), jnp.float32),
        pltpu.SemaphoreType.DMA,
        pltpu.SemaphoreType.DMA,
    ],
)

_phase2 = pl.kernel(
    _p2_body,
    out_type=[jax.ShapeDtypeStruct((_NP,), jnp.float32),
              jax.ShapeDtypeStruct((_NP,), jnp.float32)],
    mesh=_mesh,
    compiler_params=_params,
    scratch_types=[
        [pltpu.VMEM((_RW,), jnp.int32) for _ in range(_K)],
        [pltpu.VMEM((_RW,), jnp.float32) for _ in range(_K)],
        pltpu.VMEM_SHARED((_NP,), jnp.float32),
        pltpu.SemaphoreType.DMA,
        pltpu.SemaphoreType.DMA,
    ],
)

_phase3 = pl.kernel(
    _p3_body,
    out_type=[jax.ShapeDtypeStruct((_NP,), jnp.float32),
              jax.ShapeDtypeStruct((_NP,), jnp.float32)],
    mesh=_mesh,
    compiler_params=_params,
    scratch_types=[
        [pltpu.VMEM((_RW,), jnp.int32) for _ in range(_K)],
        [pltpu.VMEM((_RW,), jnp.int32) for _ in range(_K)],
        [pltpu.VMEM((_RW,), jnp.int32) for _ in range(_K)],
        [pltpu.VMEM((_RW,), jnp.float32) for _ in range(_K)],
        [pltpu.VMEM((_RW,), jnp.float32) for _ in range(_K)],
        [pltpu.VMEM((_RW,), jnp.float32) for _ in range(_K)],
        [pltpu.VMEM((_RW,), jnp.float32) for _ in range(_K)],
        pltpu.VMEM_SHARED((_NP,), jnp.float32),
        pltpu.SemaphoreType.DMA,
        pltpu.SemaphoreType.DMA,
    ],
)

_phase4 = pl.kernel(
    _p4_body,
    out_type=jax.ShapeDtypeStruct((_NP,), jnp.float32),
    mesh=_mesh,
    compiler_params=_params,
    scratch_types=[
        pltpu.VMEM((_CP4,), jnp.float32),
        pltpu.VMEM((_CP4,), jnp.float32),
        pltpu.VMEM((_CP4,), jnp.float32),
    ],
)


def kernel(inputs, selected_edges, hidden_con, hidden_uncon, rel_emb, ws, b,
           out_w, out_b):
    vi = selected_edges[:, 1]
    vj = selected_edges[:, 2]
    rel = selected_edges[:, 3]
    ivi = selected_edges[:, 4]
    hcu = jnp.concatenate([hidden_con[0], hidden_uncon[0]], axis=1)
    zeros = jnp.zeros((_NP,), jnp.float32)
    se_flat = selected_edges.reshape(-1)
    exps = _phase1(se_flat, hcu, rel_emb, ws, b, out_w, out_b)
    d0, d1 = _phase2(exps, ivi, zeros)
    den = _phase4(d0, d1)
    o0, o1 = _phase3(exps, ivi, vi, vj, den, inputs[0], zeros)
    out = _phase4(o0, o1)
    return out[:_N].reshape(1, _N)


# speedup vs baseline: 1.1770x; 1.1770x over previous
import jax
import jax.numpy as jnp
from jax import lax
from jax.experimental import pallas as pl
from jax.experimental.pallas import tpu as pltpu
from jax.experimental.pallas import tpu_sc as plsc

_N = 10000
_NP = 10240
_E = 320000
_D = 128
_NC = 2
_NS = 16
_NW = _NC * _NS
_C1 = 80
_EPW = _E // _NW
_NCH = _EPW // _C1
_RW = 80
_NR = _E // _RW
_RPW = _NR // _NW
_K = 5
_CP4 = _NP // _NW

_mesh = plsc.VectorSubcoreMesh(core_axis_name="c", subcore_axis_name="s")
_params = pltpu.CompilerParams(needs_layout_passes=False)


def _p1_body(vi_hbm, vj_hbm, rel_hbm, hcu_hbm, re_hbm, ws_hbm, b_hbm,
             ow_hbm, ob_hbm, exps_hbm,
             viu_b0, vju_b0, r_b0, viu_b1, vju_b1, r_b1,
             idx_bufs, wsv, bv, owv, obv, acc_buf, ex_b0, ex_b1,
             sem0, sem1, idx_sems):
    cid = lax.axis_index("c")
    sid = lax.axis_index("s")
    wid = sid * _NC + cid
    pltpu.sync_copy(ws_hbm, wsv)
    pltpu.sync_copy(b_hbm, bv)
    pltpu.sync_copy(ow_hbm, owv)
    pltpu.sync_copy(ob_hbm, obv)
    base0 = wid * _EPW
    slots = ((viu_b0, vju_b0, r_b0, ex_b0, sem0),
             (viu_b1, vju_b1, r_b1, ex_b1, sem1))
    idxs = tuple(zip(idx_bufs[0], idx_bufs[1], idx_bufs[2], idx_sems))

    def fire_idx(k, ib):
        vi_b, vj_b, rel_b, semI = ib
        base = base0 + k * _C1
        pltpu.async_copy(vi_hbm.at[pl.ds(base, _C1)], vi_b, semI)
        pltpu.async_copy(vj_hbm.at[pl.ds(base, _C1)], vj_b, semI)
        pltpu.async_copy(rel_hbm.at[pl.ds(base, _C1)], rel_b, semI)

    def wait_idx(ib):
        vi_b, vj_b, rel_b, semI = ib
        pltpu.make_async_copy(vi_hbm.at[pl.ds(0, _C1)], vi_b, semI).wait()
        pltpu.make_async_copy(vi_hbm.at[pl.ds(0, _C1)], vj_b, semI).wait()
        pltpu.make_async_copy(vi_hbm.at[pl.ds(0, _C1)], rel_b, semI).wait()

    def fire_rows(slot, ib):
        viu_b, vju_b, r_b, _, sem = slot
        vi_b, vj_b, rel_b = ib[:3]
        pltpu.async_copy(hcu_hbm.at[vi_b], viu_b, sem)
        pltpu.async_copy(hcu_hbm.at[vj_b], vju_b, sem)
        pltpu.async_copy(re_hbm.at[rel_b], r_b, sem)

    def drain_rows(slot, ib):
        viu_b, vju_b, r_b, _, sem = slot
        vi_b, vj_b, rel_b = ib[:3]
        pltpu.make_async_copy(hcu_hbm.at[vi_b], viu_b, sem).wait()
        pltpu.make_async_copy(hcu_hbm.at[vj_b], vju_b, sem).wait()
        pltpu.make_async_copy(re_hbm.at[rel_b], r_b, sem).wait()

    def compute_out(k, slot):
        viu_b, vju_b, r_b, ex_b, _ = slot
        base = base0 + k * _C1
        for g in range(_D // 16):
            dsl = pl.ds(g * 16, 16)
            dsl2 = pl.ds(_D + g * 16, 16)
            w0 = wsv[0, dsl]
            w1 = wsv[1, dsl]
            w2 = wsv[2, dsl]
            w3 = wsv[3, dsl]
            w4 = wsv[4, dsl]
            w5 = wsv[5, dsl]
            w6 = wsv[6, dsl]
            w7 = wsv[7, dsl]
            bb = bv[dsl]
            ow = owv[dsl]
            ob = obv[dsl]

            def ebody(e, g=g, dsl=dsl, dsl2=dsl2, viu_b=viu_b, vju_b=vju_b,
                      r_b=r_b, w0=w0, w1=w1, w2=w2, w3=w3, w4=w4, w5=w5,
                      w6=w6, w7=w7, bb=bb, ow=ow, ob=ob):
                a = viu_b[e, dsl]
                au = viu_b[e, dsl2]
                c = vju_b[e, dsl]
                cu = vju_b[e, dsl2]
                r = r_b[e, dsl]
                p = a * (w0 + w1 * r) + au * (w4 + w5 * r)
                q = a * (w2 + w3 * r) + au * (w6 + w7 * r)
                t = c * p + cu * q + bb
                t = jnp.maximum(t, 0.0) * ow + ob
                sl = pl.ds(e * 16, 16)
                if g == 0:
                    acc_buf[sl] = t
                else:
                    acc_buf[sl] = acc_buf[sl] + t

            plsc.parallel_loop(0, _C1, unroll=4)(ebody)
        for gg in range(_C1 // 16):
            rows16 = (lax.iota(jnp.int32, 16) + (gg * 16)) * 16
            tot = jnp.zeros((16,), jnp.float32)
            for j in range(16):
                tot = tot + plsc.load_gather(acc_buf, [rows16 + j])
            ex_b[pl.ds(gg * 16, 16)] = jnp.exp(tot)
        pltpu.sync_copy(ex_b, exps_hbm.at[pl.ds(base, _C1)])

    fire_idx(0, idxs[0])
    fire_idx(1, idxs[1])
    fire_idx(2, idxs[2])
    wait_idx(idxs[0])
    fire_rows(slots[0], idxs[0])

    @pl.loop(0, (_NCH - 1) // 4)
    def _quad(p):
        k0 = 4 * p
        wait_idx(idxs[1])
        fire_rows(slots[1], idxs[1])
        @pl.when(k0 + 3 < _NCH)
        def _f0(k0=k0):
            fire_idx(k0 + 3, idxs[3])
        drain_rows(slots[0], idxs[0])
        compute_out(k0 + 0, slots[0])
        wait_idx(idxs[2])
        fire_rows(slots[0], idxs[2])
        @pl.when(k0 + 4 < _NCH)
        def _f1(k0=k0):
            fire_idx(k0 + 4, idxs[0])
        drain_rows(slots[1], idxs[1])
        compute_out(k0 + 1, slots[1])
        wait_idx(idxs[3])
        fire_rows(slots[1], idxs[3])
        @pl.when(k0 + 5 < _NCH)
        def _f2(k0=k0):
            fire_idx(k0 + 5, idxs[1])
        drain_rows(slots[0], idxs[2])
        compute_out(k0 + 2, slots[0])
        wait_idx(idxs[0])
        fire_rows(slots[0], idxs[0])
        @pl.when(k0 + 6 < _NCH)
        def _f3(k0=k0):
            fire_idx(k0 + 6, idxs[2])
        drain_rows(slots[1], idxs[3])
        compute_out(k0 + 3, slots[1])

    drain_rows(slots[0], idxs[0])
    compute_out(_NCH - 1, slots[0])


def _p2_body(exps_hbm, ivi_hbm, zeros_hbm, d0_hbm, d1_hbm,
             idx_bs, val_bs, shared, semA, semB):
    cid = lax.axis_index("c")
    sid = lax.axis_index("s")
    wid = sid * _NC + cid

    @pl.when(sid == 0)
    def _zero():
        pltpu.sync_copy(zeros_hbm, shared)

    plsc.subcore_barrier()

    @pl.loop(0, _RPW // _K)
    def _blk(bi):
        r0 = wid * _RPW + bi * _K
        sc = []
        for j in range(_K):
            base = (r0 + j) * _RW
            sc.append(pltpu.async_copy(ivi_hbm.at[pl.ds(base, _RW)],
                                       idx_bs[j], semA))
            sc.append(pltpu.async_copy(exps_hbm.at[pl.ds(base, _RW)],
                                       val_bs[j], semA))
        for cp in sc:
            cp.wait()
        cps = [pltpu.async_copy(val_bs[j], shared.at[idx_bs[j]], semB,
                                add=True) for j in range(_K)]
        for cp in cps:
            cp.wait()

    plsc.subcore_barrier()

    @pl.when(sid == 0)
    def _out():
        @pl.when(cid == 0)
        def _c0():
            pltpu.sync_copy(shared, d0_hbm)

        @pl.when(cid == 1)
        def _c1():
            pltpu.sync_copy(shared, d1_hbm)


def _p3_body(exps_hbm, ivi_hbm, vi_hbm, vj_hbm, den_hbm, inp_hbm,
             zeros_hbm, o0_hbm, o1_hbm,
             ivi_bs, vi_bs, vj_bs, ex_bs, den_bs, inp_bs, att_bs,
             shared, semA, semB):
    cid = lax.axis_index("c")
    sid = lax.axis_index("s")
    wid = sid * _NC + cid

    @pl.when(sid == 0)
    def _zero():
        pltpu.sync_copy(zeros_hbm, shared)

    plsc.subcore_barrier()

    @pl.loop(0, _RPW // _K)
    def _blk(bi):
        r0 = wid * _RPW + bi * _K
        sc = []
        for j in range(_K):
            base = (r0 + j) * _RW
            sc.append(pltpu.async_copy(ivi_hbm.at[pl.ds(base, _RW)],
                                       ivi_bs[j], semA))
            sc.append(pltpu.async_copy(vi_hbm.at[pl.ds(base, _RW)],
                                       vi_bs[j], semA))
            sc.append(pltpu.async_copy(vj_hbm.at[pl.ds(base, _RW)],
                                       vj_bs[j], semA))
            sc.append(pltpu.async_copy(exps_hbm.at[pl.ds(base, _RW)],
                                       ex_bs[j], semA))
        for cp in sc:
            cp.wait()
        gcps = []
        for j in range(_K):
            gcps.append(pltpu.async_copy(den_hbm.at[ivi_bs[j]], den_bs[j],
                                         semA))
            gcps.append(pltpu.async_copy(inp_hbm.at[vi_bs[j]], inp_bs[j],
                                         semA))
        for cp in gcps:
            cp.wait()
        for j in range(_K):
            for g in range(_RW // 16):
                dsl = pl.ds(g * 16, 16)
                att_bs[j][dsl] = (ex_bs[j][dsl] * inp_bs[j][dsl]
                                  / den_bs[j][dsl])
        scps = [pltpu.async_copy(att_bs[j], shared.at[vj_bs[j]], semB,
                                 add=True) for j in range(_K)]
        for cp in scps:
            cp.wait()

    plsc.subcore_barrier()

    @pl.when(sid == 0)
    def _out():
        @pl.when(cid == 0)
        def _c0():
            pltpu.sync_copy(shared, o0_hbm)

        @pl.when(cid == 1)
        def _c1():
            pltpu.sync_copy(shared, o1_hbm)


def _p4_body(o0_hbm, o1_hbm, out_hbm, a_b, b_b, s_b):
    cid = lax.axis_index("c")
    sid = lax.axis_index("s")
    wid = sid * _NC + cid
    base = wid * _CP4
    pltpu.sync_copy(o0_hbm.at[pl.ds(base, _CP4)], a_b)
    pltpu.sync_copy(o1_hbm.at[pl.ds(base, _CP4)], b_b)
    for g in range(_CP4 // 16):
        sl = pl.ds(g * 16, 16)
        s_b[sl] = a_b[sl] + b_b[sl]
    pltpu.sync_copy(s_b, out_hbm.at[pl.ds(base, _CP4)])


_phase1 = pl.kernel(
    _p1_body,
    out_type=jax.ShapeDtypeStruct((_E,), jnp.float32),
    mesh=_mesh,
    compiler_params=_params,
    scratch_types=[
        pltpu.VMEM((_C1, 2 * _D), jnp.float32),
        pltpu.VMEM((_C1, 2 * _D), jnp.float32),
        pltpu.VMEM((_C1, _D), jnp.float32),
        pltpu.VMEM((_C1, 2 * _D), jnp.float32),
        pltpu.VMEM((_C1, 2 * _D), jnp.float32),
        pltpu.VMEM((_C1, _D), jnp.float32),
        [[pltpu.VMEM((_C1,), jnp.int32) for _ in range(4)] for _ in range(3)],
        pltpu.VMEM((8, _D), jnp.float32),
        pltpu.VMEM((_D,), jnp.float32),
        pltpu.VMEM((_D,), jnp.float32),
        pltpu.VMEM((_D,), jnp.float32),
        pltpu.VMEM((_C1 * 16,), jnp.float32),
        pltpu.VMEM((_C1,), jnp.float32),
        pltpu.VMEM((_C1,), jnp.float32),
        pltpu.SemaphoreType.DMA,
        pltpu.SemaphoreType.DMA,
        [pltpu.SemaphoreType.DMA for _ in range(4)],
    ],
)

_phase2 = pl.kernel(
    _p2_body,
    out_type=[jax.ShapeDtypeStruct((_NP,), jnp.float32),
              jax.ShapeDtypeStruct((_NP,), jnp.float32)],
    mesh=_mesh,
    compiler_params=_params,
    scratch_types=[
        [pltpu.VMEM((_RW,), jnp.int32) for _ in range(_K)],
        [pltpu.VMEM((_RW,), jnp.float32) for _ in range(_K)],
        pltpu.VMEM_SHARED((_NP,), jnp.float32),
        pltpu.SemaphoreType.DMA,
        pltpu.SemaphoreType.DMA,
    ],
)

_phase3 = pl.kernel(
    _p3_body,
    out_type=[jax.ShapeDtypeStruct((_NP,), jnp.float32),
              jax.ShapeDtypeStruct((_NP,), jnp.float32)],
    mesh=_mesh,
    compiler_params=_params,
    scratch_types=[
        [pltpu.VMEM((_RW,), jnp.int32) for _ in range(_K)],
        [pltpu.VMEM((_RW,), jnp.int32) for _ in range(_K)],
        [pltpu.VMEM((_RW,), jnp.int32) for _ in range(_K)],
        [pltpu.VMEM((_RW,), jnp.float32) for _ in range(_K)],
        [pltpu.VMEM((_RW,), jnp.float32) for _ in range(_K)],
        [pltpu.VMEM((_RW,), jnp.float32) for _ in range(_K)],
        [pltpu.VMEM((_RW,), jnp.float32) for _ in range(_K)],
        pltpu.VMEM_SHARED((_NP,), jnp.float32),
        pltpu.SemaphoreType.DMA,
        pltpu.SemaphoreType.DMA,
    ],
)

_phase4 = pl.kernel(
    _p4_body,
    out_type=jax.ShapeDtypeStruct((_NP,), jnp.float32),
    mesh=_mesh,
    compiler_params=_params,
    scratch_types=[
        pltpu.VMEM((_CP4,), jnp.float32),
        pltpu.VMEM((_CP4,), jnp.float32),
        pltpu.VMEM((_CP4,), jnp.float32),
    ],
)


def kernel(inputs, selected_edges, hidden_con, hidden_uncon, rel_emb, ws, b,
           out_w, out_b):
    vi = selected_edges[:, 1]
    vj = selected_edges[:, 2]
    rel = selected_edges[:, 3]
    ivi = selected_edges[:, 4]
    hcu = jnp.concatenate([hidden_con[0], hidden_uncon[0]], axis=1)
    zeros = jnp.zeros((_NP,), jnp.float32)
    exps = _phase1(vi, vj, rel, hcu, rel_emb, ws, b, out_w, out_b)
    d0, d1 = _phase2(exps, ivi, zeros)
    den = _phase4(d0, d1)
    o0, o1 = _phase3(exps, ivi, vi, vj, den, inputs[0], zeros)
    out = _phase4(o0, o1)
    return out[:_N].reshape(1, _N)
